# baseline (speedup 1.0000x reference)
import functools

import jax
import jax.numpy as jnp
from jax.experimental import pallas as pl
from jax.experimental.pallas import tpu as pltpu

_SEQS_PER_STEP = 8
_ROW_CHUNK = 256


def _mix_kernel(T, K, O, obs_ref, start_ref, w_ref, b_ref, act_ref,
                out_ref, fin_ref, scr_ref):
    R = obs_ref.shape[0]
    KO = K * O
    S = R // T
    f32 = jnp.float32
    hi = jax.lax.Precision.HIGHEST
    h3 = jax.lax.Precision.HIGHEST

    gj = jax.lax.broadcasted_iota(jnp.int32, (KO, K), 0)
    gk = jax.lax.broadcasted_iota(jnp.int32, (KO, K), 1)
    G = (gj // O == gk).astype(f32)
    lane_mod = jax.lax.rem(
        jax.lax.broadcasted_iota(jnp.int32, (_ROW_CHUNK, KO), 1), O)

    nchunks = R // _ROW_CHUNK
    alp_rows = []
    lse_rows = []
    for c in range(nchunks):
        r0, r1 = c * _ROW_CHUNK, (c + 1) * _ROW_CHUNK
        x = obs_ref[r0:r1, :]
        lg = jnp.dot(x, w_ref[:, :], preferred_element_type=f32, precision=h3)
        lg = lg + b_ref[:, :]
        scr_ref[r0:r1, :] = lg
        m_all = jnp.max(lg, axis=1, keepdims=True)
        e = jnp.exp(lg - m_all)
        ssum = jnp.dot(e, G, preferred_element_type=f32, precision=h3)
        lse = jnp.log(ssum) + m_all
        a = act_ref[r0:r1, :]
        ohb = (lane_mod == a).astype(f32)
        av = jnp.dot(lg * ohb, G, preferred_element_type=f32, precision=h3)
        alp_rows.append(av - lse)
        lse_rows.append(lse)
    alp = jnp.concatenate(alp_rows, axis=0)
    lse_all = jnp.concatenate(lse_rows, axis=0)

    rmod = jax.lax.rem(jax.lax.broadcasted_iota(jnp.int32, (R, 1), 0), T)
    y = jnp.where(rmod >= 1,
                  jnp.concatenate([jnp.zeros((1, K), f32), alp[:R - 1, :]],
                                  axis=0),
                  jnp.zeros((R, K), f32))
    s = 1
    while s < T:
        sh = jnp.concatenate([jnp.zeros((s, K), f32), y[:R - s, :]], axis=0)
        y = y + jnp.where(rmod >= s, sh, jnp.zeros((R, K), f32))
        s *= 2

    pr = jax.lax.broadcasted_iota(jnp.int32, (R, S), 0) // T
    pc = jax.lax.broadcasted_iota(jnp.int32, (R, S), 1)
    P = (pr == pc).astype(f32)
    z = jnp.dot(P, start_ref[:, :], preferred_element_type=f32,
                precision=hi) + y
    m16 = jnp.max(z, axis=1, keepdims=True)
    lsz = jnp.log(jnp.sum(jnp.exp(z - m16), axis=1, keepdims=True)) + m16
    mlp = z - lsz

    zi = z + alp
    qr = jax.lax.broadcasted_iota(jnp.int32, (S, R), 0)
    qc = jax.lax.broadcasted_iota(jnp.int32, (S, R), 1)
    Q = (qc == qr * T + (T - 1)).astype(f32)
    fz = jnp.dot(Q, zi, preferred_element_type=f32, precision=hi)
    fm = jnp.max(fz, axis=1, keepdims=True)
    fl = jnp.log(jnp.sum(jnp.exp(fz - fm), axis=1, keepdims=True)) + fm
    fin_ref[:, :] = fz - fl

    cmix = mlp - lse_all
    for c in range(nchunks):
        r0, r1 = c * _ROW_CHUNK, (c + 1) * _ROW_CHUNK
        lg = scr_ref[r0:r1, :]
        M = None
        for k in range(K):
            wk = lg[:, k * O:(k + 1) * O] + cmix[r0:r1, k:k + 1]
            M = wk if M is None else jnp.maximum(M, wk)
        acc = jnp.zeros_like(M)
        for k in range(K):
            wk = lg[:, k * O:(k + 1) * O] + cmix[r0:r1, k:k + 1]
            acc = acc + jnp.exp(wk - M)
        out_ref[r0:r1, :] = jnp.log(acc) + M


def kernel(obs_flat, start_mixture_logprobs, W, b, actions, seq_lens):
    ns = seq_lens.shape[0]
    Bsz, D = obs_flat.shape
    T = Bsz // ns
    K, _, O = W.shape
    KO = K * O
    S = _SEQS_PER_STEP
    R = S * T
    grid = ns // S
    wm = jnp.transpose(W, (1, 0, 2)).reshape(D, KO)
    bm = b.reshape(1, KO)
    act = actions.reshape(Bsz, 1)

    body = functools.partial(_mix_kernel, T, K, O)
    out_shapes = (jax.ShapeDtypeStruct((Bsz, O), jnp.float32),
                  jax.ShapeDtypeStruct((ns, K), jnp.float32))
    model_out, fin = pl.pallas_call(
        body,
        grid=(grid,),
        in_specs=[
            pl.BlockSpec((R, D), lambda s: (s, 0)),
            pl.BlockSpec((S, K), lambda s: (s, 0)),
            pl.BlockSpec((D, KO), lambda s: (0, 0)),
            pl.BlockSpec((1, KO), lambda s: (0, 0)),
            pl.BlockSpec((R, 1), lambda s: (s, 0)),
        ],
        out_specs=(
            pl.BlockSpec((R, O), lambda s: (s, 0)),
            pl.BlockSpec((S, K), lambda s: (s, 0)),
        ),
        out_shape=out_shapes,
        scratch_shapes=[pltpu.VMEM((R, KO), jnp.float32)],
        compiler_params=pltpu.CompilerParams(
            dimension_semantics=("parallel",)),
    )(obs_flat, start_mixture_logprobs, wm, bm, act)
    return (model_out, fin)


# speedup vs baseline: 1.6067x; 1.6067x over previous
import functools

import jax
import jax.numpy as jnp
from jax.experimental import pallas as pl
from jax.experimental.pallas import tpu as pltpu

_SEQS_PER_STEP = 8
_ROW_CHUNK = 256


def _mix_kernel(T, K, O, obs_ref, start_ref, w_ref, b_ref, act_ref,
                out_ref, fin_ref, scr_ref):
    R = obs_ref.shape[0]
    KO = K * O
    S = R // T
    f32 = jnp.float32
    hi = jax.lax.Precision.HIGHEST
    h3 = jax.lax.Precision.HIGHEST
    lo = jax.lax.Precision.DEFAULT

    gj = jax.lax.broadcasted_iota(jnp.int32, (KO, K), 0)
    gk = jax.lax.broadcasted_iota(jnp.int32, (KO, K), 1)
    G = (gj // O == gk).astype(f32)
    lane_mod = jax.lax.rem(
        jax.lax.broadcasted_iota(jnp.int32, (_ROW_CHUNK, KO), 1), O)

    nchunks = R // _ROW_CHUNK
    alp_rows = []
    lse_rows = []
    for c in range(nchunks):
        r0, r1 = c * _ROW_CHUNK, (c + 1) * _ROW_CHUNK
        x = obs_ref[r0:r1, :]
        lg = jnp.dot(x, w_ref[:, :], preferred_element_type=f32, precision=h3)
        lg = lg + b_ref[:, :]
        scr_ref[r0:r1, :] = lg
        e = jnp.exp(lg)
        ssum = jnp.dot(e, G, preferred_element_type=f32, precision=lo)
        lse = jnp.log(ssum)
        a = act_ref[r0:r1, :]
        ohb = (lane_mod == a).astype(f32)
        msel = lg * ohb
        mh = msel.astype(jnp.bfloat16).astype(f32)
        ml = msel - mh
        av = (jnp.dot(mh, G, preferred_element_type=f32, precision=lo)
              + jnp.dot(ml, G, preferred_element_type=f32, precision=lo))
        alp_rows.append(av - lse)
        lse_rows.append(lse)
    alp = jnp.concatenate(alp_rows, axis=0)
    lse_all = jnp.concatenate(lse_rows, axis=0)

    rmod = jax.lax.rem(jax.lax.broadcasted_iota(jnp.int32, (R, 1), 0), T)
    y = jnp.where(rmod >= 1,
                  jnp.concatenate([jnp.zeros((1, K), f32), alp[:R - 1, :]],
                                  axis=0),
                  jnp.zeros((R, K), f32))
    s = 1
    while s < T:
        sh = jnp.concatenate([jnp.zeros((s, K), f32), y[:R - s, :]], axis=0)
        y = y + jnp.where(rmod >= s, sh, jnp.zeros((R, K), f32))
        s *= 2

    pr = jax.lax.broadcasted_iota(jnp.int32, (R, S), 0) // T
    pc = jax.lax.broadcasted_iota(jnp.int32, (R, S), 1)
    P = (pr == pc).astype(f32)
    z = jnp.dot(P, start_ref[:, :], preferred_element_type=f32,
                precision=lo) + y
    m16 = jnp.max(z, axis=1, keepdims=True)
    lsz = jnp.log(jnp.sum(jnp.exp(z - m16), axis=1, keepdims=True)) + m16
    mlp = z - lsz

    zi = z + alp
    qr = jax.lax.broadcasted_iota(jnp.int32, (S, R), 0)
    qc = jax.lax.broadcasted_iota(jnp.int32, (S, R), 1)
    Q = (qc == qr * T + (T - 1)).astype(f32)
    fz = jnp.dot(Q, zi, preferred_element_type=f32, precision=hi)
    fm = jnp.max(fz, axis=1, keepdims=True)
    fl = jnp.log(jnp.sum(jnp.exp(fz - fm), axis=1, keepdims=True)) + fm
    fin_ref[:, :] = fz - fl

    cmix = mlp - lse_all
    for c in range(nchunks):
        r0, r1 = c * _ROW_CHUNK, (c + 1) * _ROW_CHUNK
        lg = scr_ref[r0:r1, :]
        M = None
        for k in range(K):
            wk = lg[:, k * O:(k + 1) * O] + cmix[r0:r1, k:k + 1]
            M = wk if M is None else jnp.maximum(M, wk)
        acc = jnp.zeros_like(M)
        for k in range(K):
            wk = lg[:, k * O:(k + 1) * O] + cmix[r0:r1, k:k + 1]
            acc = acc + jnp.exp(wk - M)
        out_ref[r0:r1, :] = jnp.log(acc) + M


def kernel(obs_flat, start_mixture_logprobs, W, b, actions, seq_lens):
    ns = seq_lens.shape[0]
    Bsz, D = obs_flat.shape
    T = Bsz // ns
    K, _, O = W.shape
    KO = K * O
    S = _SEQS_PER_STEP
    R = S * T
    grid = ns // S
    wm = jnp.transpose(W, (1, 0, 2)).reshape(D, KO)
    bm = b.reshape(1, KO)
    act = actions.reshape(Bsz, 1)

    body = functools.partial(_mix_kernel, T, K, O)
    out_shapes = (jax.ShapeDtypeStruct((Bsz, O), jnp.float32),
                  jax.ShapeDtypeStruct((ns, K), jnp.float32))
    model_out, fin = pl.pallas_call(
        body,
        grid=(grid,),
        in_specs=[
            pl.BlockSpec((R, D), lambda s: (s, 0)),
            pl.BlockSpec((S, K), lambda s: (s, 0)),
            pl.BlockSpec((D, KO), lambda s: (0, 0)),
            pl.BlockSpec((1, KO), lambda s: (0, 0)),
            pl.BlockSpec((R, 1), lambda s: (s, 0)),
        ],
        out_specs=(
            pl.BlockSpec((R, O), lambda s: (s, 0)),
            pl.BlockSpec((S, K), lambda s: (s, 0)),
        ),
        out_shape=out_shapes,
        scratch_shapes=[pltpu.VMEM((R, KO), jnp.float32)],
        compiler_params=pltpu.CompilerParams(
            dimension_semantics=("parallel",)),
    )(obs_flat, start_mixture_logprobs, wm, bm, act)
    return (model_out, fin)


# speedup vs baseline: 1.7332x; 1.0788x over previous
import functools

import jax
import jax.numpy as jnp
from jax.experimental import pallas as pl
from jax.experimental.pallas import tpu as pltpu

_SEQS_PER_STEP = 8
_ROW_CHUNK = 256


def _mix_kernel(T, K, O, obs_ref, start_ref, w_ref, b_ref, act_ref,
                out_ref, fin_ref, scr_ref):
    R = obs_ref.shape[0]
    KO = K * O
    S = R // T
    f32 = jnp.float32
    hi = jax.lax.Precision.HIGHEST
    h3 = jax.lax.Precision.HIGHEST
    lo = jax.lax.Precision.DEFAULT

    gj = jax.lax.broadcasted_iota(jnp.int32, (KO, K), 0)
    gk = jax.lax.broadcasted_iota(jnp.int32, (KO, K), 1)
    G = (gj // O == gk).astype(f32)
    lane_mod = jax.lax.rem(
        jax.lax.broadcasted_iota(jnp.int32, (_ROW_CHUNK, KO), 1), O)

    nchunks = R // _ROW_CHUNK
    wfull = w_ref[:, :]
    wh = wfull.astype(jnp.bfloat16).astype(f32)
    wl = wfull - wh
    alp_rows = []
    lse_rows = []
    for c in range(nchunks):
        r0, r1 = c * _ROW_CHUNK, (c + 1) * _ROW_CHUNK
        x = obs_ref[r0:r1, :]
        xh = x.astype(jnp.bfloat16).astype(f32)
        xl = x - xh
        lg = (jnp.dot(xh, wh, preferred_element_type=f32, precision=lo)
              + jnp.dot(xh, wl, preferred_element_type=f32, precision=lo)
              + jnp.dot(xl, wh, preferred_element_type=f32, precision=lo))
        lg = lg + b_ref[:, :]
        scr_ref[r0:r1, :] = lg
        e = jnp.exp(lg)
        ssum = jnp.dot(e, G, preferred_element_type=f32, precision=lo)
        lse = jnp.log(ssum)
        a = act_ref[r0:r1, :]
        ohb = (lane_mod == a).astype(f32)
        msel = lg * ohb
        mh = msel.astype(jnp.bfloat16).astype(f32)
        ml = msel - mh
        av = (jnp.dot(mh, G, preferred_element_type=f32, precision=lo)
              + jnp.dot(ml, G, preferred_element_type=f32, precision=lo))
        alp_rows.append(av - lse)
        lse_rows.append(lse)
    alp = jnp.concatenate(alp_rows, axis=0)
    lse_all = jnp.concatenate(lse_rows, axis=0)

    rmod = jax.lax.rem(jax.lax.broadcasted_iota(jnp.int32, (R, 1), 0), T)
    y = jnp.where(rmod >= 1,
                  jnp.concatenate([jnp.zeros((1, K), f32), alp[:R - 1, :]],
                                  axis=0),
                  jnp.zeros((R, K), f32))
    s = 1
    while s < T:
        sh = jnp.concatenate([jnp.zeros((s, K), f32), y[:R - s, :]], axis=0)
        y = y + jnp.where(rmod >= s, sh, jnp.zeros((R, K), f32))
        s *= 2

    pr = jax.lax.broadcasted_iota(jnp.int32, (R, S), 0) // T
    pc = jax.lax.broadcasted_iota(jnp.int32, (R, S), 1)
    P = (pr == pc).astype(f32)
    z = jnp.dot(P, start_ref[:, :], preferred_element_type=f32,
                precision=lo) + y
    m16 = jnp.max(z, axis=1, keepdims=True)
    lsz = jnp.log(jnp.sum(jnp.exp(z - m16), axis=1, keepdims=True)) + m16
    mlp = z - lsz

    zi = z + alp
    qr = jax.lax.broadcasted_iota(jnp.int32, (S, R), 0)
    qc = jax.lax.broadcasted_iota(jnp.int32, (S, R), 1)
    Q = (qc == qr * T + (T - 1)).astype(f32)
    fz = jnp.dot(Q, zi, preferred_element_type=f32, precision=hi)
    fm = jnp.max(fz, axis=1, keepdims=True)
    fl = jnp.log(jnp.sum(jnp.exp(fz - fm), axis=1, keepdims=True)) + fm
    fin_ref[:, :] = fz - fl

    cmix = mlp - lse_all
    for c in range(nchunks):
        r0, r1 = c * _ROW_CHUNK, (c + 1) * _ROW_CHUNK
        lg = scr_ref[r0:r1, :]
        M = None
        for k in range(K):
            wk = lg[:, k * O:(k + 1) * O] + cmix[r0:r1, k:k + 1]
            M = wk if M is None else jnp.maximum(M, wk)
        acc = jnp.zeros_like(M)
        for k in range(K):
            wk = lg[:, k * O:(k + 1) * O] + cmix[r0:r1, k:k + 1]
            acc = acc + jnp.exp(wk - M)
        out_ref[r0:r1, :] = jnp.log(acc) + M


def kernel(obs_flat, start_mixture_logprobs, W, b, actions, seq_lens):
    ns = seq_lens.shape[0]
    Bsz, D = obs_flat.shape
    T = Bsz // ns
    K, _, O = W.shape
    KO = K * O
    S = _SEQS_PER_STEP
    R = S * T
    grid = ns // S
    wm = jnp.transpose(W, (1, 0, 2)).reshape(D, KO)
    bm = b.reshape(1, KO)
    act = actions.reshape(Bsz, 1)

    body = functools.partial(_mix_kernel, T, K, O)
    out_shapes = (jax.ShapeDtypeStruct((Bsz, O), jnp.float32),
                  jax.ShapeDtypeStruct((ns, K), jnp.float32))
    model_out, fin = pl.pallas_call(
        body,
        grid=(grid,),
        in_specs=[
            pl.BlockSpec((R, D), lambda s: (s, 0)),
            pl.BlockSpec((S, K), lambda s: (s, 0)),
            pl.BlockSpec((D, KO), lambda s: (0, 0)),
            pl.BlockSpec((1, KO), lambda s: (0, 0)),
            pl.BlockSpec((R, 1), lambda s: (s, 0)),
        ],
        out_specs=(
            pl.BlockSpec((R, O), lambda s: (s, 0)),
            pl.BlockSpec((S, K), lambda s: (s, 0)),
        ),
        out_shape=out_shapes,
        scratch_shapes=[pltpu.VMEM((R, KO), jnp.float32)],
        compiler_params=pltpu.CompilerParams(
            dimension_semantics=("parallel",)),
    )(obs_flat, start_mixture_logprobs, wm, bm, act)
    return (model_out, fin)


# speedup vs baseline: 3.9884x; 2.3012x over previous
import functools

import jax
import jax.numpy as jnp
from jax.experimental import pallas as pl
from jax.experimental.pallas import tpu as pltpu

_SEQS_PER_STEP = 8
_COL_CHUNK = 256
_KBLK = 4


def _mix_kernel(T, K, O, obs_ref, start_ref, wh_ref, wl_ref, act_ref,
                outT_ref, fin_ref, escr_ref):
    R = obs_ref.shape[1]
    KO = K * O
    S = R // T
    f32 = jnp.float32
    hi = jax.lax.Precision.HIGHEST
    lo = jax.lax.Precision.DEFAULT

    C = _COL_CHUNK
    nch = R // C
    iota_o = jax.lax.broadcasted_iota(jnp.int32, (O, C), 0)

    alp_chunks = []
    lse_chunks = []
    for c in range(nch):
        cs = c * C
        xc = obs_ref[:, cs:cs + C]
        xh = xc.astype(jnp.bfloat16).astype(f32)
        xl = xc - xh
        a_row = act_ref[:, cs:cs + C]
        oh = iota_o == a_row
        alp_rows = []
        lse_rows = []
        for kb in range(K // _KBLK):
            j0 = kb * _KBLK * O
            whb = wh_ref[j0:j0 + _KBLK * O, :]
            wlb = wl_ref[j0:j0 + _KBLK * O, :]
            lgb = (jnp.dot(whb, xh, preferred_element_type=f32, precision=lo)
                   + jnp.dot(whb, xl, preferred_element_type=f32, precision=lo)
                   + jnp.dot(wlb, xh, preferred_element_type=f32, precision=lo))
            eb = jnp.exp(lgb)
            escr_ref[j0:j0 + _KBLK * O, cs:cs + C] = eb
            for kk in range(_KBLK):
                o0 = kk * O
                lg_k = lgb[o0:o0 + O, :]
                e_k = eb[o0:o0 + O, :]
                ssum = jnp.sum(e_k, axis=0, keepdims=True)
                lse_k = jnp.log(ssum)
                sel_k = jnp.sum(jnp.where(oh, lg_k, 0.0), axis=0,
                                keepdims=True)
                alp_rows.append(sel_k - lse_k)
                lse_rows.append(lse_k)
        alp_chunks.append(jnp.concatenate(alp_rows, axis=0))
        lse_chunks.append(jnp.concatenate(lse_rows, axis=0))
    alpT = jnp.concatenate(alp_chunks, axis=1)
    lseT = jnp.concatenate(lse_chunks, axis=1)

    cmod = jax.lax.rem(jax.lax.broadcasted_iota(jnp.int32, (1, R), 1), T)
    zero = jnp.zeros((K, R), f32)
    y = jnp.where(cmod >= 1,
                  jnp.concatenate([jnp.zeros((K, 1), f32), alpT[:, :R - 1]],
                                  axis=1),
                  zero)
    s = 1
    while s < T:
        sh = jnp.concatenate([jnp.zeros((K, s), f32), y[:, :R - s]], axis=1)
        y = y + jnp.where(cmod >= s, sh, zero)
        s *= 2

    zT = start_ref[:, :] + y
    m = jnp.max(zT, axis=0, keepdims=True)
    lsz = jnp.log(jnp.sum(jnp.exp(zT - m), axis=0, keepdims=True)) + m
    mlpT = zT - lsz

    ziT = zT + alpT
    qc = jax.lax.broadcasted_iota(jnp.int32, (R, S), 0)
    qs = jax.lax.broadcasted_iota(jnp.int32, (R, S), 1)
    Qm = (qc == qs * T + (T - 1)).astype(f32)
    fzT = jnp.dot(ziT, Qm, preferred_element_type=f32, precision=hi)
    fm = jnp.max(fzT, axis=0, keepdims=True)
    fl = jnp.log(jnp.sum(jnp.exp(fzT - fm), axis=0, keepdims=True)) + fm
    fin_ref[:, :] = jnp.transpose(fzT - fl)

    F = jnp.exp(mlpT - lseT)
    for c in range(nch):
        cs = c * C
        acc = None
        for k in range(K):
            ek = escr_ref[k * O:(k + 1) * O, cs:cs + C]
            fk = F[k:k + 1, cs:cs + C]
            t = ek * fk
            acc = t if acc is None else acc + t
        outT_ref[:, cs:cs + C] = jnp.log(acc)


def kernel(obs_flat, start_mixture_logprobs, W, b, actions, seq_lens):
    ns = seq_lens.shape[0]
    Bsz, D = obs_flat.shape
    T = Bsz // ns
    K, _, O = W.shape
    KO = K * O
    S = _SEQS_PER_STEP
    R = S * T
    grid = ns // S
    Daug = ((D + 1 + 7) // 8) * 8

    obsT = jnp.zeros((Daug, Bsz), jnp.float32)
    obsT = obsT.at[:D, :].set(obs_flat.T)
    obsT = obsT.at[D, :].set(1.0)
    wt = jnp.transpose(W, (0, 2, 1)).reshape(KO, D)
    w_aug = jnp.zeros((KO, Daug), jnp.float32)
    w_aug = w_aug.at[:, :D].set(wt)
    w_aug = w_aug.at[:, D].set(b.reshape(KO))
    wh = w_aug.astype(jnp.bfloat16).astype(jnp.float32)
    wl = w_aug - wh
    startT = jnp.repeat(start_mixture_logprobs.T, T, axis=1)
    actT = actions.reshape(1, Bsz)

    body = functools.partial(_mix_kernel, T, K, O)
    out_shapes = (jax.ShapeDtypeStruct((O, Bsz), jnp.float32),
                  jax.ShapeDtypeStruct((ns, K), jnp.float32))
    outT, fin = pl.pallas_call(
        body,
        grid=(grid,),
        in_specs=[
            pl.BlockSpec((Daug, R), lambda s: (0, s)),
            pl.BlockSpec((K, R), lambda s: (0, s)),
            pl.BlockSpec((KO, Daug), lambda s: (0, 0)),
            pl.BlockSpec((KO, Daug), lambda s: (0, 0)),
            pl.BlockSpec((1, R), lambda s: (0, s)),
        ],
        out_specs=(
            pl.BlockSpec((O, R), lambda s: (0, s)),
            pl.BlockSpec((S, K), lambda s: (s, 0)),
        ),
        out_shape=out_shapes,
        scratch_shapes=[pltpu.VMEM((KO, R), jnp.float32)],
        compiler_params=pltpu.CompilerParams(
            dimension_semantics=("parallel",)),
    )(obsT, startT, wh, wl, actT)
    return (outT.T, fin)


# speedup vs baseline: 5.5609x; 1.3942x over previous
import functools

import jax
import jax.numpy as jnp
from jax.experimental import pallas as pl
from jax.experimental.pallas import tpu as pltpu

_SEQS_PER_STEP = 8
_COL_CHUNK = 256
_KBLK = 4


def _mix_kernel(T, K, O, obs_ref, start_ref, wh_ref, wl_ref, b_ref, act_ref,
                out_ref, fin_ref, escr_ref):
    R = obs_ref.shape[0]
    KO = K * O
    S = R // T
    dnums = (((1,), (1,)), ((), ()))
    f32 = jnp.float32
    hi = jax.lax.Precision.HIGHEST
    lo = jax.lax.Precision.DEFAULT

    C = _COL_CHUNK
    nch = R // C
    iota_o = jax.lax.broadcasted_iota(jnp.int32, (O, C), 0)

    alp_chunks = []
    lse_chunks = []
    for c in range(nch):
        cs = c * C
        xc = obs_ref[cs:cs + C, :]
        xh = xc.astype(jnp.bfloat16).astype(f32)
        xl = xc - xh
        a_row = act_ref[:, cs:cs + C]
        oh = iota_o == a_row
        alp_rows = []
        lse_rows = []
        for kb in range(K // _KBLK):
            j0 = kb * _KBLK * O
            whb = wh_ref[j0:j0 + _KBLK * O, :]
            wlb = wl_ref[j0:j0 + _KBLK * O, :]
            bb = b_ref[j0:j0 + _KBLK * O, :]
            lgb = (jax.lax.dot_general(whb, xh, dnums,
                                       preferred_element_type=f32,
                                       precision=lo)
                   + jax.lax.dot_general(whb, xl, dnums,
                                         preferred_element_type=f32,
                                         precision=lo)
                   + jax.lax.dot_general(wlb, xh, dnums,
                                         preferred_element_type=f32,
                                         precision=lo)
                   + bb)
            eb = jnp.exp(lgb)
            escr_ref[j0:j0 + _KBLK * O, cs:cs + C] = eb
            for kk in range(_KBLK):
                o0 = kk * O
                lg_k = lgb[o0:o0 + O, :]
                e_k = eb[o0:o0 + O, :]
                ssum = jnp.sum(e_k, axis=0, keepdims=True)
                lse_k = jnp.log(ssum)
                sel_k = jnp.sum(jnp.where(oh, lg_k, 0.0), axis=0,
                                keepdims=True)
                alp_rows.append(sel_k - lse_k)
                lse_rows.append(lse_k)
        alp_chunks.append(jnp.concatenate(alp_rows, axis=0))
        lse_chunks.append(jnp.concatenate(lse_rows, axis=0))
    alpT = jnp.concatenate(alp_chunks, axis=1)
    lseT = jnp.concatenate(lse_chunks, axis=1)

    cmod = jax.lax.rem(jax.lax.broadcasted_iota(jnp.int32, (1, R), 1), T)
    zero = jnp.zeros((K, R), f32)
    y = jnp.where(cmod >= 1,
                  jnp.concatenate([jnp.zeros((K, 1), f32), alpT[:, :R - 1]],
                                  axis=1),
                  zero)
    s = 1
    while s < T:
        sh = jnp.concatenate([jnp.zeros((K, s), f32), y[:, :R - s]], axis=1)
        y = y + jnp.where(cmod >= s, sh, zero)
        s *= 2

    zT = start_ref[:, :] + y
    m = jnp.max(zT, axis=0, keepdims=True)
    lsz = jnp.log(jnp.sum(jnp.exp(zT - m), axis=0, keepdims=True)) + m
    mlpT = zT - lsz

    ziT = zT + alpT
    qc = jax.lax.broadcasted_iota(jnp.int32, (R, S), 0)
    qs = jax.lax.broadcasted_iota(jnp.int32, (R, S), 1)
    Qm = (qc == qs * T + (T - 1)).astype(f32)
    fzT = jnp.dot(ziT, Qm, preferred_element_type=f32, precision=hi)
    fm = jnp.max(fzT, axis=0, keepdims=True)
    fl = jnp.log(jnp.sum(jnp.exp(fzT - fm), axis=0, keepdims=True)) + fm
    fin_ref[:, :] = jnp.transpose(fzT - fl)

    F = jnp.exp(mlpT - lseT)
    for c in range(nch):
        cs = c * C
        acc = None
        for k in range(K):
            ek = escr_ref[k * O:(k + 1) * O, cs:cs + C]
            fk = F[k:k + 1, cs:cs + C]
            t = ek * fk
            acc = t if acc is None else acc + t
        out_ref[cs:cs + C, :] = jnp.transpose(jnp.log(acc))


def kernel(obs_flat, start_mixture_logprobs, W, b, actions, seq_lens):
    ns = seq_lens.shape[0]
    Bsz, D = obs_flat.shape
    T = Bsz // ns
    K, _, O = W.shape
    KO = K * O
    S = _SEQS_PER_STEP
    R = S * T
    grid = ns // S
    wt = jnp.transpose(W, (0, 2, 1)).reshape(KO, D)
    wh = wt.astype(jnp.bfloat16).astype(jnp.float32)
    wl = wt - wh
    bt = b.reshape(KO, 1)
    startT = jnp.repeat(start_mixture_logprobs.T, T, axis=1)
    actT = actions.reshape(1, Bsz)

    body = functools.partial(_mix_kernel, T, K, O)
    out_shapes = (jax.ShapeDtypeStruct((Bsz, O), jnp.float32),
                  jax.ShapeDtypeStruct((ns, K), jnp.float32))
    model_out, fin = pl.pallas_call(
        body,
        grid=(grid,),
        in_specs=[
            pl.BlockSpec((R, D), lambda s: (s, 0)),
            pl.BlockSpec((K, R), lambda s: (0, s)),
            pl.BlockSpec((KO, D), lambda s: (0, 0)),
            pl.BlockSpec((KO, D), lambda s: (0, 0)),
            pl.BlockSpec((KO, 1), lambda s: (0, 0)),
            pl.BlockSpec((1, R), lambda s: (0, s)),
        ],
        out_specs=(
            pl.BlockSpec((R, O), lambda s: (s, 0)),
            pl.BlockSpec((S, K), lambda s: (s, 0)),
        ),
        out_shape=out_shapes,
        scratch_shapes=[pltpu.VMEM((KO, R), jnp.float32)],
        compiler_params=pltpu.CompilerParams(
            dimension_semantics=("parallel",)),
    )(obs_flat, startT, wh, wl, bt, actT)
    return (model_out, fin)
